# baseline (speedup 1.0000x reference)
import functools

import jax
import jax.numpy as jnp
from jax import lax
from jax.experimental import pallas as pl
from jax.experimental.pallas import tpu as pltpu
from jax.experimental.pallas import tpu_sc as plsc

_BATCH = 16384
_NW = 16
_BPW = _BATCH // _NW

_mesh = plsc.VectorSubcoreMesh(
    core_axis_name="c", subcore_axis_name="s", num_cores=1
)


@functools.partial(
    pl.kernel,
    mesh=_mesh,
    out_type=jax.ShapeDtypeStruct((_BATCH,), jnp.float32),
    scratch_types=[
        pltpu.VMEM((_BPW,), jnp.int32),
        pltpu.VMEM((_BPW,), jnp.float32),
        pltpu.SemaphoreType.DMA,
    ],
)
def _sc_gather(id_hbm, data_hbm, out_hbm, idx_v, vals_v, sem):
    base = lax.axis_index("s") * _BPW
    pltpu.sync_copy(id_hbm.at[pl.ds(base, _BPW)], idx_v)
    pltpu.async_copy(data_hbm.at[idx_v], vals_v, sem).wait()
    pltpu.sync_copy(vals_v, out_hbm.at[pl.ds(base, _BPW)])


def kernel(id, data):
    return _sc_gather(id.astype(jnp.int32), data)


# speedup vs baseline: 1.0163x; 1.0163x over previous
import functools

import jax
import jax.numpy as jnp
from jax import lax
from jax.experimental import pallas as pl
from jax.experimental.pallas import tpu as pltpu
from jax.experimental.pallas import tpu_sc as plsc

_BATCH = 16384
_NW = 16
_BPW = _BATCH // _NW
_CHUNKS = (64, 448, 448, 64)
_OFFS = (0, 64, 512, 960)
_NCHUNK = len(_CHUNKS)

_mesh = plsc.VectorSubcoreMesh(
    core_axis_name="c", subcore_axis_name="s", num_cores=1
)


@functools.partial(
    pl.kernel,
    mesh=_mesh,
    out_type=jax.ShapeDtypeStruct((_BATCH,), jnp.float32),
    scratch_types=[
        pltpu.VMEM((_BPW,), jnp.int32),
        pltpu.VMEM((_BPW,), jnp.float32),
    ]
    + [pltpu.SemaphoreType.DMA] * (3 * _NCHUNK),
)
def _sc_gather(id_hbm, data_hbm, out_hbm, idx_v, vals_v, *sems):
    base = lax.axis_index("s") * _BPW
    stages = []
    for j in range(_NCHUNK):
        o, c = _OFFS[j], _CHUNKS[j]
        stages.append(
            pltpu.async_copy(
                id_hbm.at[pl.ds(base + o, c)],
                idx_v.at[pl.ds(o, c)],
                sems[j],
            )
        )
    gathers = []
    for j in range(_NCHUNK):
        o, c = _OFFS[j], _CHUNKS[j]
        stages[j].wait()
        gathers.append(
            pltpu.async_copy(
                data_hbm.at[idx_v.at[pl.ds(o, c)]],
                vals_v.at[pl.ds(o, c)],
                sems[_NCHUNK + j],
            )
        )
    writebacks = []
    for j in range(_NCHUNK):
        o, c = _OFFS[j], _CHUNKS[j]
        gathers[j].wait()
        writebacks.append(
            pltpu.async_copy(
                vals_v.at[pl.ds(o, c)],
                out_hbm.at[pl.ds(base + o, c)],
                sems[2 * _NCHUNK + j],
            )
        )
    for cp in writebacks:
        cp.wait()


def kernel(id, data):
    return _sc_gather(id.astype(jnp.int32), data)
